# baseline (speedup 1.0000x reference)
import jax
import jax.numpy as jnp
from jax.experimental import pallas as pl

N = 10000
F = 128
BM = 400


def _support_kernel(h_ref, w_ref, s_ref):
    s_ref[...] = jnp.dot(h_ref[...], w_ref[...],
                         preferred_element_type=jnp.float32)


def _layer1_kernel(adj_ref, s_ref, b1_ref, b2_ref, t_ref):
    acc = jnp.dot(adj_ref[...], s_ref[...],
                  preferred_element_type=jnp.float32)
    g = jnp.tanh(acc[:, :F] + b1_ref[...])
    z = jax.nn.sigmoid(acc[:, F:] + b2_ref[...])
    t_ref[...] = g * z


def _layer2_kernel(adj_ref, u_ref, bmu_ref, bls_ref, mu_ref, ls_ref):
    acc = jnp.dot(adj_ref[...], u_ref[...],
                  preferred_element_type=jnp.float32)
    mu_ref[...] = acc[:, :F] + bmu_ref[...]
    ls_ref[...] = acc[:, F:] + bls_ref[...]


def _support(h, wcat):
    return pl.pallas_call(
        _support_kernel,
        out_shape=jax.ShapeDtypeStruct((N, 2 * F), jnp.float32),
    )(h, wcat)


def kernel(x, adj, W1, b1, W2, b2, Wmu, bmu, Wls, bls):
    wc1 = jnp.concatenate([W1, W2], axis=1)
    wc2 = jnp.concatenate([Wmu, Wls], axis=1)
    b1r = b1.reshape(1, F)
    b2r = b2.reshape(1, F)
    bmur = bmu.reshape(1, F)
    blsr = bls.reshape(1, F)

    grid = (N // BM,)
    adj_spec = pl.BlockSpec((BM, N), lambda i: (i, 0))
    rhs_spec = pl.BlockSpec((N, 2 * F), lambda i: (0, 0))
    bias_spec = pl.BlockSpec((1, F), lambda i: (0, 0))
    out_spec = pl.BlockSpec((BM, F), lambda i: (i, 0))

    s12 = _support(x, wc1)
    t = pl.pallas_call(
        _layer1_kernel,
        grid=grid,
        in_specs=[adj_spec, rhs_spec, bias_spec, bias_spec],
        out_specs=out_spec,
        out_shape=jax.ShapeDtypeStruct((N, F), jnp.float32),
    )(adj, s12, b1r, b2r)

    u = _support(t, wc2)
    mu, ls = pl.pallas_call(
        _layer2_kernel,
        grid=grid,
        in_specs=[adj_spec, rhs_spec, bias_spec, bias_spec],
        out_specs=[out_spec, out_spec],
        out_shape=[jax.ShapeDtypeStruct((N, F), jnp.float32),
                   jax.ShapeDtypeStruct((N, F), jnp.float32)],
    )(adj, u, bmur, blsr)
    return (mu, ls)


# speedup vs baseline: 1.0099x; 1.0099x over previous
import jax
import jax.numpy as jnp
from jax.experimental import pallas as pl

N = 10000
F = 128
BM = 400


def _support_kernel(h_ref, w_ref, s_ref):
    s_ref[...] = jnp.dot(
        h_ref[...].astype(jnp.bfloat16), w_ref[...],
        preferred_element_type=jnp.float32).astype(jnp.bfloat16)


def _layer1_kernel(adj_ref, s_ref, b1_ref, b2_ref, t_ref):
    acc = jnp.dot(adj_ref[...].astype(jnp.bfloat16), s_ref[...],
                  preferred_element_type=jnp.float32)
    g = jnp.tanh(acc[:, :F] + b1_ref[...])
    z = jax.nn.sigmoid(acc[:, F:] + b2_ref[...])
    t_ref[...] = (g * z).astype(jnp.bfloat16)


def _layer2_kernel(adj_ref, u_ref, bmu_ref, bls_ref, mu_ref, ls_ref):
    acc = jnp.dot(adj_ref[...].astype(jnp.bfloat16), u_ref[...],
                  preferred_element_type=jnp.float32)
    mu_ref[...] = acc[:, :F] + bmu_ref[...]
    ls_ref[...] = acc[:, F:] + bls_ref[...]


def _support(h, wcat):
    return pl.pallas_call(
        _support_kernel,
        out_shape=jax.ShapeDtypeStruct((N, 2 * F), jnp.bfloat16),
    )(h, wcat)


def kernel(x, adj, W1, b1, W2, b2, Wmu, bmu, Wls, bls):
    wc1 = jnp.concatenate([W1, W2], axis=1).astype(jnp.bfloat16)
    wc2 = jnp.concatenate([Wmu, Wls], axis=1).astype(jnp.bfloat16)
    b1r = b1.reshape(1, F)
    b2r = b2.reshape(1, F)
    bmur = bmu.reshape(1, F)
    blsr = bls.reshape(1, F)

    grid = (N // BM,)
    adj_spec = pl.BlockSpec((BM, N), lambda i: (i, 0))
    rhs_spec = pl.BlockSpec((N, 2 * F), lambda i: (0, 0))
    bias_spec = pl.BlockSpec((1, F), lambda i: (0, 0))
    out_spec = pl.BlockSpec((BM, F), lambda i: (i, 0))

    s12 = _support(x, wc1)
    t = pl.pallas_call(
        _layer1_kernel,
        grid=grid,
        in_specs=[adj_spec, rhs_spec, bias_spec, bias_spec],
        out_specs=out_spec,
        out_shape=jax.ShapeDtypeStruct((N, F), jnp.bfloat16),
    )(adj, s12, b1r, b2r)

    u = _support(t, wc2)
    mu, ls = pl.pallas_call(
        _layer2_kernel,
        grid=grid,
        in_specs=[adj_spec, rhs_spec, bias_spec, bias_spec],
        out_specs=[out_spec, out_spec],
        out_shape=[jax.ShapeDtypeStruct((N, F), jnp.float32),
                   jax.ShapeDtypeStruct((N, F), jnp.float32)],
    )(adj, u, bmur, blsr)
    return (mu, ls)


# speedup vs baseline: 1.0673x; 1.0569x over previous
import jax
import jax.numpy as jnp
from jax.experimental import pallas as pl
from jax.experimental.pallas import tpu as pltpu

N = 10000
F = 128
BM = 400


def _layer1_kernel(adj_ref, x_ref, w_ref, b1_ref, b2_ref, t_ref, s_ref):
    @pl.when(pl.program_id(0) == 0)
    def _():
        s_ref[...] = jnp.dot(
            x_ref[...].astype(jnp.bfloat16), w_ref[...],
            preferred_element_type=jnp.float32).astype(jnp.bfloat16)

    acc = jnp.dot(adj_ref[...].astype(jnp.bfloat16), s_ref[...],
                  preferred_element_type=jnp.float32)
    g = jnp.tanh(acc[:, :F] + b1_ref[...])
    z = jax.nn.sigmoid(acc[:, F:] + b2_ref[...])
    t_ref[...] = (g * z).astype(jnp.bfloat16)


def _layer2_kernel(adj_ref, t_ref, w_ref, bmu_ref, bls_ref,
                   mu_ref, ls_ref, u_ref):
    @pl.when(pl.program_id(0) == 0)
    def _():
        u_ref[...] = jnp.dot(
            t_ref[...], w_ref[...],
            preferred_element_type=jnp.float32).astype(jnp.bfloat16)

    acc = jnp.dot(adj_ref[...].astype(jnp.bfloat16), u_ref[...],
                  preferred_element_type=jnp.float32)
    mu_ref[...] = acc[:, :F] + bmu_ref[...]
    ls_ref[...] = acc[:, F:] + bls_ref[...]


def kernel(x, adj, W1, b1, W2, b2, Wmu, bmu, Wls, bls):
    wc1 = jnp.concatenate([W1, W2], axis=1).astype(jnp.bfloat16)
    wc2 = jnp.concatenate([Wmu, Wls], axis=1).astype(jnp.bfloat16)
    b1r = b1.reshape(1, F)
    b2r = b2.reshape(1, F)
    bmur = bmu.reshape(1, F)
    blsr = bls.reshape(1, F)

    grid = (N // BM,)
    adj_spec = pl.BlockSpec((BM, N), lambda i: (i, 0))
    full_spec = pl.BlockSpec((N, F), lambda i: (0, 0))
    w_spec = pl.BlockSpec((F, 2 * F), lambda i: (0, 0))
    bias_spec = pl.BlockSpec((1, F), lambda i: (0, 0))
    out_spec = pl.BlockSpec((BM, F), lambda i: (i, 0))
    s_scratch = pltpu.VMEM((N, 2 * F), jnp.bfloat16)

    t = pl.pallas_call(
        _layer1_kernel,
        grid=grid,
        in_specs=[adj_spec, full_spec, w_spec, bias_spec, bias_spec],
        out_specs=out_spec,
        out_shape=jax.ShapeDtypeStruct((N, F), jnp.bfloat16),
        scratch_shapes=[s_scratch],
    )(adj, x, wc1, b1r, b2r)

    mu, ls = pl.pallas_call(
        _layer2_kernel,
        grid=grid,
        in_specs=[adj_spec, full_spec, w_spec, bias_spec, bias_spec],
        out_specs=[out_spec, out_spec],
        out_shape=[jax.ShapeDtypeStruct((N, F), jnp.float32),
                   jax.ShapeDtypeStruct((N, F), jnp.float32)],
        scratch_shapes=[s_scratch],
    )(adj, t, wc2, bmur, blsr)
    return (mu, ls)
